# baseline (speedup 1.0000x reference)
import jax
import jax.numpy as jnp
from jax import lax
from jax.experimental import pallas as pl
from jax.experimental.pallas import tpu as pltpu, tpu_sc as plsc

N_ROWS = 4096
N_COLS = 164
N_HAND = 84
N_OUT = 128
N_TILES = 16
ROWS_PER_TILE = N_ROWS // N_TILES
OUT_PER_TILE = N_OUT // N_TILES
_CHUNKS = (0, 16, 32, 48, 64, 80, 96, 112, 128, 144, 148)
_NAN_LIM = 0x7F800000


def _sc_body(data_hbm, out_hbm,
             blk_v, loclist_v, cnt16_v, cbuf_v, lists_v, obuf_v, cntb_v,
             jlo_v, jhi_v, rows_lo, rows_hi, outbuf_v,
             counts_sh, lists_sh, sem, sem2):
    wid = lax.axis_index("s")
    iota = lax.iota(jnp.int32, 16)

    qrt = ROWS_PER_TILE // 4
    ca = pltpu.async_copy(data_hbm.at[pl.ds(wid * qrt, qrt // 2)],
                          blk_v.at[pl.ds(0, qrt // 2)], sem)
    cb = pltpu.async_copy(data_hbm.at[pl.ds(wid * qrt + qrt // 2, qrt // 2)],
                          blk_v.at[pl.ds(qrt // 2, qrt // 2)], sem2)

    def g_body(g, cnt):
        rowidx = g * 16 + iota
        q = rowidx >> 2
        sub = (rowidx & 3) * N_COLS
        accs = [jnp.zeros((16,), jnp.float32) for _ in range(4)]
        for c in range(N_HAND):
            v = plsc.load_gather(blk_v, [q, sub + c])
            mag = plsc.bitcast(v, jnp.int32) & jnp.int32(0x7FFFFFFF)
            absv = plsc.bitcast(mag, jnp.float32)
            accs[c % 4] = accs[c % 4] + jnp.where(mag > _NAN_LIM, 0.0, absv)
        acc = (accs[0] + accs[1]) + (accs[2] + accs[3])
        m = acc != 0.0
        tvec = wid * ROWS_PER_TILE + rowidx
        plsc.store_compressed(loclist_v.at[pl.ds(cnt, 16)], tvec, mask=m)
        return cnt + jnp.sum(m.astype(jnp.int32))

    ca.wait()
    cnt = lax.fori_loop(0, ROWS_PER_TILE // 32, g_body, jnp.int32(0))
    cb.wait()
    cnt = lax.fori_loop(ROWS_PER_TILE // 32, ROWS_PER_TILE // 16, g_body,
                        cnt)

    cnt16_v[...] = jnp.full((16,), 0, jnp.int32) + cnt
    pltpu.sync_copy(cnt16_v, counts_sh.at[wid])

    @pl.when(cnt < ROWS_PER_TILE)
    def _publish():
        pltpu.sync_copy(loclist_v.at[pl.ds(0, ROWS_PER_TILE)],
                        lists_sh.at[wid])

    plsc.subcore_barrier()

    pltpu.sync_copy(counts_sh, cbuf_v)

    cvec = plsc.load_gather(cbuf_v, [iota, jnp.zeros((16,), jnp.int32)])
    nn = jnp.sum(cvec)

    @pl.when(nn < N_ROWS)
    def _fetch_lists():
        pltpu.sync_copy(lists_sh, lists_v)

    ends = plsc.cumsum(cvec)
    offs = ends - cvec
    n = nn

    in_w = jnp.maximum(n, N_OUT)
    in_w_f = in_w.astype(jnp.float32)
    scale = in_w_f * (1.0 / N_OUT)
    i_f = (wid * OUT_PER_TILE + iota).astype(jnp.float32)
    src = (i_f + 0.5) * scale - 0.5
    src = jnp.clip(src, 0.0, in_w_f - 1.0)
    lo = src.astype(jnp.int32)
    hi = jnp.minimum(lo + 1, in_w - 1)
    frac = src - lo.astype(jnp.float32)

    s_lo = jnp.zeros((16,), jnp.int32)
    s_hi = jnp.zeros((16,), jnp.int32)
    for w in range(N_TILES):
        e = ends[w]
        s_lo = s_lo + (lo >= e).astype(jnp.int32)
        s_hi = s_hi + (hi >= e).astype(jnp.int32)
    s_lo = jnp.minimum(s_lo, N_TILES - 1)
    s_hi = jnp.minimum(s_hi, N_TILES - 1)

    obuf_v[...] = offs
    cntb_v[...] = cvec
    loc_lo = jnp.clip(lo - plsc.load_gather(obuf_v, [s_lo]), 0,
                      ROWS_PER_TILE - 1)
    loc_hi = jnp.clip(hi - plsc.load_gather(obuf_v, [s_hi]), 0,
                      ROWS_PER_TILE - 1)
    full_lo = plsc.load_gather(cntb_v, [s_lo]) == ROWS_PER_TILE
    full_hi = plsc.load_gather(cntb_v, [s_hi]) == ROWS_PER_TILE
    t_lo = jnp.where(full_lo, s_lo * ROWS_PER_TILE + loc_lo,
                     plsc.load_gather(lists_v, [s_lo, loc_lo]))
    t_hi = jnp.where(full_hi, s_hi * ROWS_PER_TILE + loc_hi,
                     plsc.load_gather(lists_v, [s_hi, loc_hi]))
    t_lo = jnp.clip(t_lo, 0, N_ROWS - 1)
    t_hi = jnp.clip(t_hi, 0, N_ROWS - 1)
    plsc.store_scatter(jlo_v, [iota], t_lo >> 2, mask=iota < OUT_PER_TILE)
    plsc.store_scatter(jhi_v, [iota], t_hi >> 2, mask=iota < OUT_PER_TILE)
    sub_lo = (t_lo & 3) * N_COLS
    sub_hi = (t_hi & 3) * N_COLS

    c1 = pltpu.async_copy(data_hbm.at[jlo_v], rows_lo, sem)
    c2 = pltpu.async_copy(data_hbm.at[jhi_v], rows_hi, sem2)
    c1.wait()
    c2.wait()

    wlo = jnp.where(lo < n, 1.0 - frac, 0.0)
    whi = jnp.where(hi < n, frac, 0.0)

    for k in range(OUT_PER_TILE):
        a = wlo[k]
        b = whi[k]
        slo = sub_lo[k]
        shi = sub_hi[k]
        for off in _CHUNKS:
            vl = rows_lo[k, pl.ds(slo + off, 16)]
            vh = rows_hi[k, pl.ds(shi + off, 16)]
            ml = plsc.bitcast(vl, jnp.int32) & jnp.int32(0x7FFFFFFF)
            mh = plsc.bitcast(vh, jnp.int32) & jnp.int32(0x7FFFFFFF)
            vl = jnp.where(ml > _NAN_LIM, 0.0, vl)
            vh = jnp.where(mh > _NAN_LIM, 0.0, vh)
            outbuf_v[k, pl.ds(off, 16)] = a * vl + b * vh

    pltpu.sync_copy(outbuf_v, out_hbm.at[pl.ds(wid * OUT_PER_TILE,
                                               OUT_PER_TILE)])


_SCRATCH = [
    pltpu.VMEM((ROWS_PER_TILE // 4, 4 * N_COLS), jnp.float32),
    pltpu.VMEM((ROWS_PER_TILE + 16,), jnp.int32),
    pltpu.VMEM((16,), jnp.int32),
    pltpu.VMEM((N_TILES, 16), jnp.int32),
    pltpu.VMEM((N_TILES, ROWS_PER_TILE), jnp.int32),
    pltpu.VMEM((16,), jnp.int32),
    pltpu.VMEM((16,), jnp.int32),
    pltpu.VMEM((OUT_PER_TILE,), jnp.int32),
    pltpu.VMEM((OUT_PER_TILE,), jnp.int32),
    pltpu.VMEM((OUT_PER_TILE, 4 * N_COLS), jnp.float32),
    pltpu.VMEM((OUT_PER_TILE, 4 * N_COLS), jnp.float32),
    pltpu.VMEM((OUT_PER_TILE, N_COLS), jnp.float32),
    pltpu.VMEM_SHARED((N_TILES, 16), jnp.int32),
    pltpu.VMEM_SHARED((N_TILES, ROWS_PER_TILE), jnp.int32),
    pltpu.SemaphoreType.DMA,
    pltpu.SemaphoreType.DMA,
]


def _build(interpret=False):
    mesh = plsc.VectorSubcoreMesh(core_axis_name="c", subcore_axis_name="s",
                                  num_cores=1, num_subcores=N_TILES)
    return pl.kernel(
        _sc_body,
        out_type=jax.ShapeDtypeStruct((N_OUT, N_COLS), jnp.float32),
        mesh=mesh,
        scratch_types=_SCRATCH,
        compiler_params=pltpu.CompilerParams(use_tc_tiling_on_sc=False,
                                             needs_layout_passes=False),
        interpret=interpret,
    )


_PREPROC = _build()


@jax.jit
def kernel(data0):
    data2d = data0.reshape(N_ROWS // 4, 4 * N_COLS)
    return _PREPROC(data2d)


# speedup vs baseline: 1.0843x; 1.0843x over previous
import jax
import jax.numpy as jnp
from jax import lax
from jax.experimental import pallas as pl
from jax.experimental.pallas import tpu as pltpu, tpu_sc as plsc

N_ROWS = 4096
N_COLS = 164
N_HAND = 84
N_OUT = 128
N_TILES = 16
ROWS_PER_TILE = N_ROWS // N_TILES
OUT_PER_TILE = N_OUT // N_TILES
_CHUNKS = (0, 16, 32, 48, 64, 80, 96, 112, 128, 144, 148)
_NAN_LIM = 0x7F800000


def _sc_body(data_hbm, out_hbm,
             blk_v, loclist_v, cnt16_v, cbuf_v, lists_v, obuf_v, cntb_v,
             jlo_v, jhi_v, rows_lo, rows_hi, outbuf_v,
             counts_sh, lists_sh, sem, sem2):
    wid = lax.axis_index("s")
    iota = lax.iota(jnp.int32, 16)

    pltpu.sync_copy(data_hbm.at[pl.ds(wid * (ROWS_PER_TILE // 4),
                                      ROWS_PER_TILE // 4)], blk_v)

    def g_body(g, cnt):
        rowidx = g * 16 + iota
        q = rowidx >> 2
        sub = (rowidx & 3) * N_COLS

        def c_chunk(ci, accs):
            a0, a1, a2, a3 = accs
            accs = [a0, a1, a2, a3]
            base = ci * 12
            for u in range(12):
                v = plsc.load_gather(blk_v, [q, sub + (base + u)])
                mag = plsc.bitcast(v, jnp.int32) & jnp.int32(0x7FFFFFFF)
                absv = plsc.bitcast(mag, jnp.float32)
                accs[u % 4] = accs[u % 4] + jnp.where(mag > _NAN_LIM, 0.0,
                                                      absv)
            return tuple(accs)

        z = jnp.zeros((16,), jnp.float32)
        accs = lax.fori_loop(0, N_HAND // 12, c_chunk, (z, z, z, z))
        acc = (accs[0] + accs[1]) + (accs[2] + accs[3])
        m = acc != 0.0
        tvec = wid * ROWS_PER_TILE + rowidx
        plsc.store_compressed(loclist_v.at[pl.ds(cnt, 16)], tvec, mask=m)
        return cnt + jnp.sum(m.astype(jnp.int32))

    cnt = lax.fori_loop(0, ROWS_PER_TILE // 16, g_body, jnp.int32(0))

    cnt16_v[...] = jnp.full((16,), 0, jnp.int32) + cnt
    pltpu.sync_copy(cnt16_v, counts_sh.at[wid])

    @pl.when(cnt < ROWS_PER_TILE)
    def _publish():
        pltpu.sync_copy(loclist_v.at[pl.ds(0, ROWS_PER_TILE)],
                        lists_sh.at[wid])

    plsc.subcore_barrier()

    pltpu.sync_copy(counts_sh, cbuf_v)

    cvec = plsc.load_gather(cbuf_v, [iota, jnp.zeros((16,), jnp.int32)])
    nn = jnp.sum(cvec)

    @pl.when(nn < N_ROWS)
    def _fetch_lists():
        pltpu.sync_copy(lists_sh, lists_v)

    ends = plsc.cumsum(cvec)
    offs = ends - cvec
    n = nn

    in_w = jnp.maximum(n, N_OUT)
    in_w_f = in_w.astype(jnp.float32)
    scale = in_w_f * (1.0 / N_OUT)
    i_f = (wid * OUT_PER_TILE + iota).astype(jnp.float32)
    src = (i_f + 0.5) * scale - 0.5
    src = jnp.clip(src, 0.0, in_w_f - 1.0)
    lo = src.astype(jnp.int32)
    hi = jnp.minimum(lo + 1, in_w - 1)
    frac = src - lo.astype(jnp.float32)

    s_lo = jnp.zeros((16,), jnp.int32)
    s_hi = jnp.zeros((16,), jnp.int32)
    for w in range(N_TILES):
        e = ends[w]
        s_lo = s_lo + (lo >= e).astype(jnp.int32)
        s_hi = s_hi + (hi >= e).astype(jnp.int32)
    s_lo = jnp.minimum(s_lo, N_TILES - 1)
    s_hi = jnp.minimum(s_hi, N_TILES - 1)

    obuf_v[...] = offs
    cntb_v[...] = cvec
    loc_lo = jnp.clip(lo - plsc.load_gather(obuf_v, [s_lo]), 0,
                      ROWS_PER_TILE - 1)
    loc_hi = jnp.clip(hi - plsc.load_gather(obuf_v, [s_hi]), 0,
                      ROWS_PER_TILE - 1)
    full_lo = plsc.load_gather(cntb_v, [s_lo]) == ROWS_PER_TILE
    full_hi = plsc.load_gather(cntb_v, [s_hi]) == ROWS_PER_TILE
    t_lo = jnp.where(full_lo, s_lo * ROWS_PER_TILE + loc_lo,
                     plsc.load_gather(lists_v, [s_lo, loc_lo]))
    t_hi = jnp.where(full_hi, s_hi * ROWS_PER_TILE + loc_hi,
                     plsc.load_gather(lists_v, [s_hi, loc_hi]))
    t_lo = jnp.clip(t_lo, 0, N_ROWS - 1)
    t_hi = jnp.clip(t_hi, 0, N_ROWS - 1)
    plsc.store_scatter(jlo_v, [iota], t_lo >> 2, mask=iota < OUT_PER_TILE)
    plsc.store_scatter(jhi_v, [iota], t_hi >> 2, mask=iota < OUT_PER_TILE)
    sub_lo = (t_lo & 3) * N_COLS
    sub_hi = (t_hi & 3) * N_COLS

    c1 = pltpu.async_copy(data_hbm.at[jlo_v], rows_lo, sem)
    c2 = pltpu.async_copy(data_hbm.at[jhi_v], rows_hi, sem2)
    c1.wait()
    c2.wait()

    wlo = jnp.where(lo < n, 1.0 - frac, 0.0)
    whi = jnp.where(hi < n, frac, 0.0)

    slo_k = [sub_lo[k] for k in range(OUT_PER_TILE)]
    shi_k = [sub_hi[k] for k in range(OUT_PER_TILE)]
    wlo_k = [wlo[k] for k in range(OUT_PER_TILE)]
    whi_k = [whi[k] for k in range(OUT_PER_TILE)]

    def blend_chunk(off):
        for k in range(OUT_PER_TILE):
            vl = rows_lo[k, pl.ds(slo_k[k] + off, 16)]
            vh = rows_hi[k, pl.ds(shi_k[k] + off, 16)]
            ml = plsc.bitcast(vl, jnp.int32) & jnp.int32(0x7FFFFFFF)
            mh = plsc.bitcast(vh, jnp.int32) & jnp.int32(0x7FFFFFFF)
            vl = jnp.where(ml > _NAN_LIM, 0.0, vl)
            vh = jnp.where(mh > _NAN_LIM, 0.0, vh)
            outbuf_v[k, pl.ds(off, 16)] = wlo_k[k] * vl + whi_k[k] * vh

    def blend_body(c, carry):
        blend_chunk(c * 16)
        return carry

    lax.fori_loop(0, 10, blend_body, jnp.int32(0))
    blend_chunk(148)

    pltpu.sync_copy(outbuf_v, out_hbm.at[pl.ds(wid * OUT_PER_TILE,
                                               OUT_PER_TILE)])


_SCRATCH = [
    pltpu.VMEM((ROWS_PER_TILE // 4, 4 * N_COLS), jnp.float32),
    pltpu.VMEM((ROWS_PER_TILE + 16,), jnp.int32),
    pltpu.VMEM((16,), jnp.int32),
    pltpu.VMEM((N_TILES, 16), jnp.int32),
    pltpu.VMEM((N_TILES, ROWS_PER_TILE), jnp.int32),
    pltpu.VMEM((16,), jnp.int32),
    pltpu.VMEM((16,), jnp.int32),
    pltpu.VMEM((OUT_PER_TILE,), jnp.int32),
    pltpu.VMEM((OUT_PER_TILE,), jnp.int32),
    pltpu.VMEM((OUT_PER_TILE, 4 * N_COLS), jnp.float32),
    pltpu.VMEM((OUT_PER_TILE, 4 * N_COLS), jnp.float32),
    pltpu.VMEM((OUT_PER_TILE, N_COLS), jnp.float32),
    pltpu.VMEM_SHARED((N_TILES, 16), jnp.int32),
    pltpu.VMEM_SHARED((N_TILES, ROWS_PER_TILE), jnp.int32),
    pltpu.SemaphoreType.DMA,
    pltpu.SemaphoreType.DMA,
]


def _build(interpret=False):
    mesh = plsc.VectorSubcoreMesh(core_axis_name="c", subcore_axis_name="s",
                                  num_cores=1, num_subcores=N_TILES)
    return pl.kernel(
        _sc_body,
        out_type=jax.ShapeDtypeStruct((N_OUT, N_COLS), jnp.float32),
        mesh=mesh,
        scratch_types=_SCRATCH,
        compiler_params=pltpu.CompilerParams(use_tc_tiling_on_sc=False,
                                             needs_layout_passes=False),
        interpret=interpret,
    )


_PREPROC = _build()


@jax.jit
def kernel(data0):
    data2d = data0.reshape(N_ROWS // 4, 4 * N_COLS)
    return _PREPROC(data2d)


# speedup vs baseline: 1.0923x; 1.0073x over previous
import jax
import jax.numpy as jnp
from jax import lax
from jax.experimental import pallas as pl
from jax.experimental.pallas import tpu as pltpu, tpu_sc as plsc

N_ROWS = 4096
N_COLS = 164
N_HAND = 84
N_OUT = 128
N_TILES = 16
ROWS_PER_TILE = N_ROWS // N_TILES
OUT_PER_TILE = N_OUT // N_TILES
_CHUNKS = (0, 16, 32, 48, 64, 80, 96, 112, 128, 144, 148)
_NAN_LIM = 0x7F800000


def _sc_body(data_hbm, out_hbm,
             blk_v, loclist_v, cnt16_v, cbuf_v, lists_v, obuf_v, cntb_v,
             jlo_v, jhi_v, rows_lo, rows_hi, outbuf_v,
             counts_sh, lists_sh, sem, sem2):
    wid = lax.axis_index("s")
    iota = lax.iota(jnp.int32, 16)

    pltpu.sync_copy(data_hbm.at[pl.ds(wid * (ROWS_PER_TILE // 4),
                                      ROWS_PER_TILE // 4)], blk_v)

    def g_body(g, cnt):
        rowidx = g * 16 + iota
        q = rowidx >> 2
        sub = (rowidx & 3) * N_COLS

        def c_chunk(ci, accs):
            a0, a1, a2, a3 = accs
            accs = [a0, a1, a2, a3]
            base = ci * 12
            for u in range(12):
                v = plsc.load_gather(blk_v, [q, sub + (base + u)])
                mag = plsc.bitcast(v, jnp.int32) & jnp.int32(0x7FFFFFFF)
                absv = plsc.bitcast(mag, jnp.float32)
                accs[u % 4] = accs[u % 4] + jnp.where(mag > _NAN_LIM, 0.0,
                                                      absv)
            return tuple(accs)

        z = jnp.zeros((16,), jnp.float32)
        accs = lax.fori_loop(0, N_HAND // 12, c_chunk, (z, z, z, z))
        acc = (accs[0] + accs[1]) + (accs[2] + accs[3])
        m = acc != 0.0
        tvec = wid * ROWS_PER_TILE + rowidx
        plsc.store_compressed(loclist_v.at[pl.ds(cnt, 16)], tvec, mask=m)
        return cnt + plsc.all_reduce_population_count(m)[0]

    cnt = lax.fori_loop(0, ROWS_PER_TILE // 16, g_body, jnp.int32(0))

    cnt16_v[...] = jnp.full((16,), 0, jnp.int32) + cnt
    pltpu.sync_copy(cnt16_v, counts_sh.at[wid])

    @pl.when(cnt < ROWS_PER_TILE)
    def _publish():
        pltpu.sync_copy(loclist_v.at[pl.ds(0, ROWS_PER_TILE)],
                        lists_sh.at[wid])

    plsc.subcore_barrier()

    pltpu.sync_copy(counts_sh, cbuf_v)

    cvec = plsc.load_gather(cbuf_v, [iota, jnp.zeros((16,), jnp.int32)])
    nn = jnp.sum(cvec)

    @pl.when(nn < N_ROWS)
    def _fetch_lists():
        pltpu.sync_copy(lists_sh, lists_v)

    ends = plsc.cumsum(cvec)
    offs = ends - cvec
    n = nn

    in_w = jnp.maximum(n, N_OUT)
    in_w_f = in_w.astype(jnp.float32)
    scale = in_w_f * (1.0 / N_OUT)
    i_f = (wid * OUT_PER_TILE + iota).astype(jnp.float32)
    src = (i_f + 0.5) * scale - 0.5
    src = jnp.clip(src, 0.0, in_w_f - 1.0)
    lo = src.astype(jnp.int32)
    hi = jnp.minimum(lo + 1, in_w - 1)
    frac = src - lo.astype(jnp.float32)

    def take16(vec, idx):
        return lax.gather(
            vec, idx[:, None],
            dimension_numbers=lax.GatherDimensionNumbers(
                offset_dims=(), collapsed_slice_dims=(0,),
                start_index_map=(0,)),
            slice_sizes=(1,),
            mode=lax.GatherScatterMode.PROMISE_IN_BOUNDS)

    def search(j):
        s = jnp.zeros((16,), jnp.int32)
        for b in (8, 4, 2, 1):
            e = take16(ends, s + (b - 1))
            s = s + b * (e <= j).astype(jnp.int32)
        return s

    s_lo = search(lo)
    s_hi = search(hi)

    obuf_v[...] = offs
    cntb_v[...] = cvec
    loc_lo = jnp.clip(lo - plsc.load_gather(obuf_v, [s_lo]), 0,
                      ROWS_PER_TILE - 1)
    loc_hi = jnp.clip(hi - plsc.load_gather(obuf_v, [s_hi]), 0,
                      ROWS_PER_TILE - 1)
    full_lo = plsc.load_gather(cntb_v, [s_lo]) == ROWS_PER_TILE
    full_hi = plsc.load_gather(cntb_v, [s_hi]) == ROWS_PER_TILE
    t_lo = jnp.where(full_lo, s_lo * ROWS_PER_TILE + loc_lo,
                     plsc.load_gather(lists_v, [s_lo, loc_lo]))
    t_hi = jnp.where(full_hi, s_hi * ROWS_PER_TILE + loc_hi,
                     plsc.load_gather(lists_v, [s_hi, loc_hi]))
    t_lo = jnp.clip(t_lo, 0, N_ROWS - 1)
    t_hi = jnp.clip(t_hi, 0, N_ROWS - 1)
    plsc.store_scatter(jlo_v, [iota], t_lo >> 2, mask=iota < OUT_PER_TILE)
    plsc.store_scatter(jhi_v, [iota], t_hi >> 2, mask=iota < OUT_PER_TILE)
    sub_lo = (t_lo & 3) * N_COLS
    sub_hi = (t_hi & 3) * N_COLS

    c1 = pltpu.async_copy(data_hbm.at[jlo_v], rows_lo, sem)
    c2 = pltpu.async_copy(data_hbm.at[jhi_v], rows_hi, sem2)
    c1.wait()
    c2.wait()

    wlo = jnp.where(lo < n, 1.0 - frac, 0.0)
    whi = jnp.where(hi < n, frac, 0.0)

    slo_k = [sub_lo[k] for k in range(OUT_PER_TILE)]
    shi_k = [sub_hi[k] for k in range(OUT_PER_TILE)]
    wlo_k = [wlo[k] for k in range(OUT_PER_TILE)]
    whi_k = [whi[k] for k in range(OUT_PER_TILE)]

    def blend_chunk(off):
        for k in range(OUT_PER_TILE):
            vl = rows_lo[k, pl.ds(slo_k[k] + off, 16)]
            vh = rows_hi[k, pl.ds(shi_k[k] + off, 16)]
            ml = plsc.bitcast(vl, jnp.int32) & jnp.int32(0x7FFFFFFF)
            mh = plsc.bitcast(vh, jnp.int32) & jnp.int32(0x7FFFFFFF)
            vl = jnp.where(ml > _NAN_LIM, 0.0, vl)
            vh = jnp.where(mh > _NAN_LIM, 0.0, vh)
            outbuf_v[k, pl.ds(off, 16)] = wlo_k[k] * vl + whi_k[k] * vh

    def blend_body(c, carry):
        blend_chunk(c * 16)
        return carry

    lax.fori_loop(0, 10, blend_body, jnp.int32(0))
    blend_chunk(148)

    pltpu.sync_copy(outbuf_v, out_hbm.at[pl.ds(wid * OUT_PER_TILE,
                                               OUT_PER_TILE)])


_SCRATCH = [
    pltpu.VMEM((ROWS_PER_TILE // 4, 4 * N_COLS), jnp.float32),
    pltpu.VMEM((ROWS_PER_TILE + 16,), jnp.int32),
    pltpu.VMEM((16,), jnp.int32),
    pltpu.VMEM((N_TILES, 16), jnp.int32),
    pltpu.VMEM((N_TILES, ROWS_PER_TILE), jnp.int32),
    pltpu.VMEM((16,), jnp.int32),
    pltpu.VMEM((16,), jnp.int32),
    pltpu.VMEM((OUT_PER_TILE,), jnp.int32),
    pltpu.VMEM((OUT_PER_TILE,), jnp.int32),
    pltpu.VMEM((OUT_PER_TILE, 4 * N_COLS), jnp.float32),
    pltpu.VMEM((OUT_PER_TILE, 4 * N_COLS), jnp.float32),
    pltpu.VMEM((OUT_PER_TILE, N_COLS), jnp.float32),
    pltpu.VMEM_SHARED((N_TILES, 16), jnp.int32),
    pltpu.VMEM_SHARED((N_TILES, ROWS_PER_TILE), jnp.int32),
    pltpu.SemaphoreType.DMA,
    pltpu.SemaphoreType.DMA,
]


def _build(interpret=False):
    mesh = plsc.VectorSubcoreMesh(core_axis_name="c", subcore_axis_name="s",
                                  num_cores=1, num_subcores=N_TILES)
    return pl.kernel(
        _sc_body,
        out_type=jax.ShapeDtypeStruct((N_OUT, N_COLS), jnp.float32),
        mesh=mesh,
        scratch_types=_SCRATCH,
        compiler_params=pltpu.CompilerParams(use_tc_tiling_on_sc=False,
                                             needs_layout_passes=False),
        interpret=interpret,
    )


_PREPROC = _build()


@jax.jit
def kernel(data0):
    data2d = data0.reshape(N_ROWS // 4, 4 * N_COLS)
    return _PREPROC(data2d)
